# baseline (speedup 1.0000x reference)
import jax
import jax.numpy as jnp
from jax.experimental import pallas as pl
from jax.experimental.pallas import tpu as pltpu

_B = 64
_N = 100000
_GRID = 8
_CHUNK = 12800
_CAP = 1e-6


def _body(p_ref, m_ref, loss_ref, tgt_ref, non_ref, acc_all, acc_tgt, acc_cnt):
    i = pl.program_id(0)

    @pl.when(i == 0)
    def _init():
        acc_all[...] = jnp.zeros_like(acc_all)
        acc_tgt[...] = jnp.zeros_like(acc_tgt)
        acc_cnt[...] = jnp.zeros_like(acc_cnt)

    def _accum(edge):
        p = p_ref[...]
        m = m_ref[...]
        if edge:
            col = jax.lax.broadcasted_iota(jnp.int32, (_B, _CHUNK), 1)
            valid = col < (_N - (_GRID - 1) * _CHUNK)
            m = jnp.logical_and(m, valid)
            t = jnp.where(m, p, jnp.where(valid, 1.0 - p, 1.0))
        else:
            t = jnp.where(m, p, 1.0 - p)
        l = jnp.log(jnp.maximum(t, _CAP))
        lm = jnp.where(m, l, 0.0)
        mf = m.astype(jnp.float32)
        acc_all[...] += jnp.sum(l, axis=1, keepdims=True)
        acc_tgt[...] += jnp.sum(lm, axis=1, keepdims=True)
        acc_cnt[...] += jnp.sum(mf, axis=1, keepdims=True)

    @pl.when(i < _GRID - 1)
    def _main():
        _accum(False)

    @pl.when(i == _GRID - 1)
    def _edge():
        _accum(True)

    @pl.when(i == _GRID - 1)
    def _fin():
        n_tgt = acc_cnt[...]
        s_tgt = acc_tgt[...]
        s_non = acc_all[...] - s_tgt
        n_non = float(_N) - n_tgt
        lt = -(s_tgt / n_tgt)
        ln = -(s_non / n_non)
        loss_tgt = jnp.sum(lt) * (1.0 / _B)
        loss_non = jnp.sum(ln) * (1.0 / _B)
        loss = loss_tgt + loss_non
        loss_ref[...] = jnp.full((8, 128), loss, jnp.float32)
        tgt_ref[...] = jnp.full((8, 128), loss_tgt, jnp.float32)
        non_ref[...] = jnp.full((8, 128), loss_non, jnp.float32)


def kernel(prob_pred, mask_gt):
    outs = pl.pallas_call(
        _body,
        grid=(_GRID,),
        in_specs=[
            pl.BlockSpec((_B, _CHUNK), lambda i: (0, i)),
            pl.BlockSpec((_B, _CHUNK), lambda i: (0, i)),
        ],
        out_specs=[
            pl.BlockSpec((8, 128), lambda i: (0, 0)),
            pl.BlockSpec((8, 128), lambda i: (0, 0)),
            pl.BlockSpec((8, 128), lambda i: (0, 0)),
        ],
        out_shape=[jax.ShapeDtypeStruct((8, 128), jnp.float32)] * 3,
        scratch_shapes=[pltpu.VMEM((_B, 1), jnp.float32)] * 3,
        compiler_params=pltpu.CompilerParams(
            dimension_semantics=("arbitrary",)
        ),
    )(prob_pred, mask_gt)
    loss, lt, ln = outs
    return (loss[0, 0], lt[0, 0], ln[0, 0])
